# Initial kernel scaffold; baseline (speedup 1.0000x reference)
import jax
import jax.numpy as jnp
from jax.experimental import pallas as pl


def kernel(x, pos, edge_index_cov, edge_attr_cov, edge_index_ncov, W_bp, b_bp, Wc1, bc1, Wc2, bc2, Wn1, bn1, Wn2, bn2, Wuc, buc, Wun, bun):
    raise NotImplementedError("write your pallas kernel here")


# speedup vs baseline: 2.6793x; 2.6793x over previous
import functools

import jax
import jax.numpy as jnp
from jax import lax
from jax.experimental import pallas as pl
from jax.experimental.pallas import tpu as pltpu
from jax.experimental.pallas import tpu_sc as plsc

F32 = jnp.float32
D = 128
NW = 32
L = 16
CHUNK = 128
NP = 10240
DUMMY = 10100


def _sc_mesh():
    return plsc.VectorSubcoreMesh(core_axis_name="c", subcore_axis_name="s",
                                  num_cores=2, num_subcores=16)


def _prep_body(x_ref, wcat_ref, wbp_ref, wc1c_ref, bbp_ref, bc1_ref,
               t1c_ref, t2c_ref, t1n_ref, t2n_ref, wattr_ref, biasc_ref):
    P = jnp.dot(x_ref[...], wcat_ref[...], preferred_element_type=F32)
    t1c_ref[...] = P[:, 0:128]
    t2c_ref[...] = P[:, 128:256]
    t1n_ref[...] = P[:, 256:384]
    t2n_ref[...] = P[:, 384:512]

    @pl.when(pl.program_id(0) == 0)
    def _():
        wattr_ref[...] = jnp.dot(wbp_ref[...], wc1c_ref[...],
                                 preferred_element_type=F32)
        biasc_ref[...] = jnp.dot(bbp_ref[...], wc1c_ref[...],
                                 preferred_element_type=F32) + bc1_ref[...]


def _prep(x, wcat, wbp, wc1c, bbp_row, bc1_row):
    n = x.shape[0]
    rb = 1000
    return pl.pallas_call(
        _prep_body,
        grid=(n // rb,),
        in_specs=[
            pl.BlockSpec((rb, D), lambda i: (i, 0)),
            pl.BlockSpec((D, 512), lambda i: (0, 0)),
            pl.BlockSpec((16, D), lambda i: (0, 0)),
            pl.BlockSpec((D, D), lambda i: (0, 0)),
            pl.BlockSpec((1, D), lambda i: (0, 0)),
            pl.BlockSpec((1, D), lambda i: (0, 0)),
        ],
        out_specs=[
            pl.BlockSpec((rb, D), lambda i: (i, 0)),
            pl.BlockSpec((rb, D), lambda i: (i, 0)),
            pl.BlockSpec((rb, D), lambda i: (i, 0)),
            pl.BlockSpec((rb, D), lambda i: (i, 0)),
            pl.BlockSpec((16, D), lambda i: (0, 0)),
            pl.BlockSpec((1, D), lambda i: (0, 0)),
        ],
        out_shape=[
            jax.ShapeDtypeStruct((n, D), F32),
            jax.ShapeDtypeStruct((n, D), F32),
            jax.ShapeDtypeStruct((n, D), F32),
            jax.ShapeDtypeStruct((n, D), F32),
            jax.ShapeDtypeStruct((16, D), F32),
            jax.ShapeDtypeStruct((1, D), F32),
        ],
    )(x, wcat, wbp, wc1c, bbp_row, bc1_row)


def _gather(t1, t2, colp, rowp, px, py, pz, ep):
    per = ep // NW
    chunks = per // CHUNK
    n = px.shape[0]

    @functools.partial(
        pl.kernel,
        out_type=(jax.ShapeDtypeStruct((ep, D), F32),
                  jax.ShapeDtypeStruct((ep, D), F32),
                  jax.ShapeDtypeStruct((ep,), F32)),
        mesh=_sc_mesh(),
        compiler_params=pltpu.CompilerParams(needs_layout_passes=False),
        scratch_types=[
            pltpu.VMEM((CHUNK,), jnp.int32),
            pltpu.VMEM((CHUNK,), jnp.int32),
            pltpu.VMEM((CHUNK, D), F32),
            pltpu.VMEM((CHUNK, D), F32),
            pltpu.VMEM((n,), F32),
            pltpu.VMEM((n,), F32),
            pltpu.VMEM((n,), F32),
            pltpu.VMEM((CHUNK,), F32),
            pltpu.SemaphoreType.DMA,
            pltpu.SemaphoreType.DMA,
        ],
    )
    def gk(t1_h, t2_h, col_h, row_h, px_h, py_h, pz_h, g1_h, g2_h, d2_h,
           ic_v, ir_v, r1_v, r2_v, px_v, py_v, pz_v, d2_v, s1, s2):
        wid = lax.axis_index("s") * 2 + lax.axis_index("c")
        pltpu.sync_copy(px_h, px_v)
        pltpu.sync_copy(py_h, py_v)
        pltpu.sync_copy(pz_h, pz_v)

        def body(j, carry):
            base = wid * per + j * CHUNK
            pltpu.sync_copy(col_h.at[pl.ds(base, CHUNK)], ic_v)
            pltpu.sync_copy(row_h.at[pl.ds(base, CHUNK)], ir_v)
            c1 = pltpu.async_copy(t1_h.at[ic_v], r1_v, s1)
            c2 = pltpu.async_copy(t2_h.at[ir_v], r2_v, s2)
            for k in range(CHUNK // L):
                ic = ic_v[pl.ds(k * L, L)]
                ir = ir_v[pl.ds(k * L, L)]
                dx = plsc.load_gather(px_v, [ic]) - plsc.load_gather(px_v, [ir])
                dy = plsc.load_gather(py_v, [ic]) - plsc.load_gather(py_v, [ir])
                dz = plsc.load_gather(pz_v, [ic]) - plsc.load_gather(pz_v, [ir])
                d2_v[pl.ds(k * L, L)] = dx * dx + dy * dy + dz * dz
            c1.wait()
            c2.wait()
            pltpu.sync_copy(r1_v, g1_h.at[pl.ds(base, CHUNK)])
            pltpu.sync_copy(r2_v, g2_h.at[pl.ds(base, CHUNK)])
            pltpu.sync_copy(d2_v, d2_h.at[pl.ds(base, CHUNK)])
            return carry

        lax.fori_loop(0, chunks, body, 0)

    return gk(t1, t2, colp, rowp, px, py, pz)


def _edge_cov_body(g1_ref, g2_ref, d2_ref, at_ref, wd_ref, wa_ref, bias_ref,
                   r_ref):
    d2 = d2_ref[...]
    dist = jnp.sqrt(d2 + 1e-12)
    dp = jnp.clip(dist, 1e-2, 50.0)
    cent = lax.broadcasted_iota(jnp.int32, (1, 64), 1).astype(F32) * (10.0 / 63.0)
    t = dp - cent
    rbf = jnp.exp(-10.0 * t * t)
    h = (g1_ref[...] + g2_ref[...]
         + jnp.dot(rbf, wd_ref[...], preferred_element_type=F32)
         + jnp.dot(at_ref[...], wa_ref[...], preferred_element_type=F32)
         + bias_ref[...])
    r_ref[...] = jnp.maximum(h, 0.0)


def _edge_cov(g1, g2, d2, attr, wd, wattr, biasc, ep):
    eb = 512
    return pl.pallas_call(
        _edge_cov_body,
        grid=(ep // eb,),
        in_specs=[
            pl.BlockSpec((eb, D), lambda i: (i, 0)),
            pl.BlockSpec((eb, D), lambda i: (i, 0)),
            pl.BlockSpec((eb, 1), lambda i: (i, 0)),
            pl.BlockSpec((eb, 16), lambda i: (i, 0)),
            pl.BlockSpec((64, D), lambda i: (0, 0)),
            pl.BlockSpec((16, D), lambda i: (0, 0)),
            pl.BlockSpec((1, D), lambda i: (0, 0)),
        ],
        out_specs=pl.BlockSpec((eb, D), lambda i: (i, 0)),
        out_shape=jax.ShapeDtypeStruct((ep, D), F32),
    )(g1, g2, d2, attr, wd, wattr, biasc)


def _edge_ncov_body(g1_ref, g2_ref, d2_ref, wn_ref, bias_ref, r_ref):
    d2 = d2_ref[...]
    dist = jnp.sqrt(d2 + 1e-12)
    dc = jnp.clip(dist, 1e-2, 50.0)
    dp2 = 1.0 / (dc * dc)
    dp6 = dp2 * dp2 * dp2
    cent = lax.broadcasted_iota(jnp.int32, (1, 64), 1).astype(F32) * (10.0 / 63.0)
    t2 = dp2 - cent
    t6 = dp6 - cent
    rbf = jnp.concatenate([jnp.exp(-10.0 * t2 * t2),
                           jnp.exp(-10.0 * t6 * t6)], axis=1)
    h = (g1_ref[...] + g2_ref[...]
         + jnp.dot(rbf, wn_ref[...], preferred_element_type=F32)
         + bias_ref[...])
    r_ref[...] = jnp.maximum(h, 0.0)


def _edge_ncov(g1, g2, d2, wn, biasn, ep):
    eb = 512
    return pl.pallas_call(
        _edge_ncov_body,
        grid=(ep // eb,),
        in_specs=[
            pl.BlockSpec((eb, D), lambda i: (i, 0)),
            pl.BlockSpec((eb, D), lambda i: (i, 0)),
            pl.BlockSpec((eb, 1), lambda i: (i, 0)),
            pl.BlockSpec((D, D), lambda i: (0, 0)),
            pl.BlockSpec((1, D), lambda i: (0, 0)),
        ],
        out_specs=pl.BlockSpec((eb, D), lambda i: (i, 0)),
        out_shape=jax.ShapeDtypeStruct((ep, D), F32),
    )(g1, g2, d2, wn, biasn)


def _scatter(r, dst, ep):
    per = ep // NW
    chunks = per // CHUNK
    zr = NP // 16

    @functools.partial(
        pl.kernel,
        out_type=(jax.ShapeDtypeStruct((2, NP, D), F32),
                  jax.ShapeDtypeStruct((2, NP), F32)),
        mesh=_sc_mesh(),
        compiler_params=pltpu.CompilerParams(needs_layout_passes=False),
        scratch_types=[
            pltpu.VMEM((CHUNK,), jnp.int32),
            pltpu.VMEM((CHUNK, D), F32),
            pltpu.VMEM((NP,), F32),
            pltpu.VMEM((16, zr), F32),
            pltpu.VMEM((zr,), F32),
            pltpu.VMEM_SHARED((NP, D), F32),
            pltpu.VMEM_SHARED((16, NP), F32),
        ],
    )
    def sk(r_h, dst_h, zm_h, m_h, cnt_h,
           idx_v, rows_v, cnt_v, cbuf_v, cz_v, accm, cstage):
        cid = lax.axis_index("c")
        sid = lax.axis_index("s")
        wid = sid * 2 + cid
        pltpu.sync_copy(zm_h, accm.at[pl.ds(sid * zr, zr)])

        def zbody(i, carry):
            cnt_v[pl.ds(i * L, L)] = jnp.zeros((L,), F32)
            return carry

        lax.fori_loop(0, NP // L, zbody, 0)
        plsc.subcore_barrier()

        def body(j, carry):
            base = wid * per + j * CHUNK
            pltpu.sync_copy(dst_h.at[pl.ds(base, CHUNK)], idx_v)
            pltpu.sync_copy(r_h.at[pl.ds(base, CHUNK)], rows_v)
            pltpu.sync_copy(rows_v, accm.at[idx_v], add=True)
            for k in range(CHUNK // L):
                idx = idx_v[pl.ds(k * L, L)]
                plsc.addupdate_scatter(cnt_v, [idx], jnp.ones((L,), F32))
            return carry

        lax.fori_loop(0, chunks, body, 0)
        pltpu.sync_copy(cnt_v, cstage.at[sid])
        plsc.subcore_barrier()
        pltpu.sync_copy(accm.at[pl.ds(sid * zr, zr)],
                        m_h.at[cid, pl.ds(sid * zr, zr)])
        pltpu.sync_copy(cstage.at[:, pl.ds(sid * zr, zr)], cbuf_v)

        def rbody(g, carry):
            acc = jnp.zeros((L,), F32)
            for rr in range(16):
                acc = acc + cbuf_v[rr, pl.ds(g * L, L)]
            cz_v[pl.ds(g * L, L)] = acc
            return carry

        lax.fori_loop(0, zr // L, rbody, 0)
        pltpu.sync_copy(cz_v, cnt_h.at[cid, pl.ds(sid * zr, zr)])

    zm = jnp.zeros((zr, D), F32)
    return sk(r, dst, zm)


def _final_body(x_ref, mc_ref, cc0_ref, cc1_ref, mn_ref, cn0_ref, cn1_ref,
                wc2_ref, bc2_ref, wn2_ref, bn2_ref,
                wuc_ref, buc_ref, wun_ref, bun_ref, o_ref):
    xb = x_ref[...]
    sc = mc_ref[0] + mc_ref[1]
    cntc = cc0_ref[...] + cc1_ref[...]
    m_cov = (jnp.dot(sc, wc2_ref[...], preferred_element_type=F32)
             + cntc * bc2_ref[...])
    sn = mn_ref[0] + mn_ref[1]
    cntn = cn0_ref[...] + cn1_ref[...]
    m_ncov = (jnp.dot(sn, wn2_ref[...], preferred_element_type=F32)
              + cntn * bn2_ref[...])
    h_cov = jnp.maximum(
        jnp.dot(xb + m_cov, wuc_ref[...], preferred_element_type=F32)
        + buc_ref[...], 0.0)
    h_ncov = jnp.maximum(
        jnp.dot(xb + m_ncov, wun_ref[...], preferred_element_type=F32)
        + bun_ref[...], 0.0)
    o_ref[...] = h_cov + h_ncov


def _final(x, mc, cc0, cc1, mn, cn0, cn1,
           wc2, bc2r, wn2, bn2r, wuc, bucr, wun, bunr):
    n = x.shape[0]
    rb = 1000
    return pl.pallas_call(
        _final_body,
        grid=(n // rb,),
        in_specs=[
            pl.BlockSpec((rb, D), lambda i: (i, 0)),
            pl.BlockSpec((2, rb, D), lambda i: (0, i, 0)),
            pl.BlockSpec((rb, 1), lambda i: (i, 0)),
            pl.BlockSpec((rb, 1), lambda i: (i, 0)),
            pl.BlockSpec((2, rb, D), lambda i: (0, i, 0)),
            pl.BlockSpec((rb, 1), lambda i: (i, 0)),
            pl.BlockSpec((rb, 1), lambda i: (i, 0)),
            pl.BlockSpec((D, D), lambda i: (0, 0)),
            pl.BlockSpec((1, D), lambda i: (0, 0)),
            pl.BlockSpec((D, D), lambda i: (0, 0)),
            pl.BlockSpec((1, D), lambda i: (0, 0)),
            pl.BlockSpec((D, D), lambda i: (0, 0)),
            pl.BlockSpec((1, D), lambda i: (0, 0)),
            pl.BlockSpec((D, D), lambda i: (0, 0)),
            pl.BlockSpec((1, D), lambda i: (0, 0)),
        ],
        out_specs=pl.BlockSpec((rb, D), lambda i: (i, 0)),
        out_shape=jax.ShapeDtypeStruct((n, D), F32),
    )(x, mc, cc0, cc1, mn, cn0, cn1, wc2, bc2r, wn2, bn2r, wuc, bucr, wun, bunr)


def _pad_idx(idx, ep, fill):
    e = idx.shape[0]
    return jnp.concatenate([idx, jnp.full((ep - e,), fill, jnp.int32)])


def kernel(x, pos, edge_index_cov, edge_attr_cov, edge_index_ncov,
           W_bp, b_bp, Wc1, bc1, Wc2, bc2, Wn1, bn1, Wn2, bn2,
           Wuc, buc, Wun, bun):
    ec = edge_index_cov.shape[1]
    en = edge_index_ncov.shape[1]
    epc = -(-ec // (NW * CHUNK)) * (NW * CHUNK)
    epn = -(-en // (NW * CHUNK)) * (NW * CHUNK)

    wcat = jnp.concatenate([Wc1[0:128], Wc1[128:256], Wn1[0:128], Wn1[128:256]],
                           axis=1)
    t1c, t2c, t1n, t2n, wattr, biasc = _prep(
        x, wcat, W_bp, Wc1[256:384], b_bp[None, :], bc1[None, :])

    px, py, pz = pos[:, 0], pos[:, 1], pos[:, 2]
    rowc = _pad_idx(edge_index_cov[0], epc, 0)
    colc = _pad_idx(edge_index_cov[1], epc, 0)
    dstc = _pad_idx(edge_index_cov[1], epc, DUMMY)
    rown = _pad_idx(edge_index_ncov[0], epn, 0)
    coln = _pad_idx(edge_index_ncov[1], epn, 0)
    dstn = _pad_idx(edge_index_ncov[1], epn, DUMMY)
    attrp = jnp.pad(edge_attr_cov, ((0, epc - ec), (0, 0)))

    gc1, gc2, d2c = _gather(t1c, t2c, colc, rowc, px, py, pz, epc)
    gn1, gn2, d2n = _gather(t1n, t2n, coln, rown, px, py, pz, epn)

    rc = _edge_cov(gc1, gc2, d2c[:, None], attrp, Wc1[384:448], wattr, biasc,
                   epc)
    rn = _edge_ncov(gn1, gn2, d2n[:, None], Wn1[256:384], bn1[None, :], epn)

    mc, cc = _scatter(rc, dstc, epc)
    mn, cn = _scatter(rn, dstn, epn)

    out = _final(x, mc, cc[0][:, None], cc[1][:, None],
                 mn, cn[0][:, None], cn[1][:, None],
                 Wc2, bc2[None, :], Wn2, bn2[None, :],
                 Wuc, buc[None, :], Wun, bun[None, :])
    return out


# speedup vs baseline: 2.6897x; 1.0039x over previous
import functools

import jax
import jax.numpy as jnp
from jax import lax
from jax.experimental import pallas as pl
from jax.experimental.pallas import tpu as pltpu
from jax.experimental.pallas import tpu_sc as plsc

F32 = jnp.float32
BF16 = jnp.bfloat16
D = 128
NW = 32
L = 16
CHUNK = 128
NP = 10240
DUMMY = 10100


def _sc_mesh():
    return plsc.VectorSubcoreMesh(core_axis_name="c", subcore_axis_name="s",
                                  num_cores=2, num_subcores=16)


def _prep_body(x_ref, wcat_ref, wbp_ref, wc1c_ref, bbp_ref, bc1_ref,
               t1c_ref, t2c_ref, t1n_ref, t2n_ref, wattr_ref, biasc_ref):
    P = jnp.dot(x_ref[...], wcat_ref[...], preferred_element_type=F32)
    t1c_ref[...] = P[:, 0:128]
    t2c_ref[...] = P[:, 128:256]
    t1n_ref[...] = P[:, 256:384]
    t2n_ref[...] = P[:, 384:512]

    @pl.when(pl.program_id(0) == 0)
    def _():
        wattr_ref[...] = jnp.dot(wbp_ref[...], wc1c_ref[...],
                                 preferred_element_type=F32)
        biasc_ref[...] = jnp.dot(bbp_ref[...], wc1c_ref[...],
                                 preferred_element_type=F32) + bc1_ref[...]


def _prep(x, wcat, wbp, wc1c, bbp_row, bc1_row):
    n = x.shape[0]
    rb = 1000
    return pl.pallas_call(
        _prep_body,
        grid=(n // rb,),
        in_specs=[
            pl.BlockSpec((rb, D), lambda i: (i, 0)),
            pl.BlockSpec((D, 512), lambda i: (0, 0)),
            pl.BlockSpec((16, D), lambda i: (0, 0)),
            pl.BlockSpec((D, D), lambda i: (0, 0)),
            pl.BlockSpec((1, D), lambda i: (0, 0)),
            pl.BlockSpec((1, D), lambda i: (0, 0)),
        ],
        out_specs=[
            pl.BlockSpec((rb, D), lambda i: (i, 0)),
            pl.BlockSpec((rb, D), lambda i: (i, 0)),
            pl.BlockSpec((rb, D), lambda i: (i, 0)),
            pl.BlockSpec((rb, D), lambda i: (i, 0)),
            pl.BlockSpec((16, D), lambda i: (0, 0)),
            pl.BlockSpec((1, D), lambda i: (0, 0)),
        ],
        out_shape=[
            jax.ShapeDtypeStruct((n, D), F32),
            jax.ShapeDtypeStruct((n, D), F32),
            jax.ShapeDtypeStruct((n, D), F32),
            jax.ShapeDtypeStruct((n, D), F32),
            jax.ShapeDtypeStruct((16, D), F32),
            jax.ShapeDtypeStruct((1, D), F32),
        ],
    )(x, wcat, wbp, wc1c, bbp_row, bc1_row)


PH = 128


def _gather(t1, t2, colp, rowp, px, py, pz, ep):
    per = ep // NW
    pairs = per // (2 * PH)
    n = px.shape[0]

    @functools.partial(
        pl.kernel,
        out_type=(jax.ShapeDtypeStruct((ep, D), F32),
                  jax.ShapeDtypeStruct((ep, D), F32),
                  jax.ShapeDtypeStruct((ep,), F32)),
        mesh=_sc_mesh(),
        compiler_params=pltpu.CompilerParams(needs_layout_passes=False),
        scratch_types=[
            pltpu.VMEM((PH,), jnp.int32),
            pltpu.VMEM((PH,), jnp.int32),
            pltpu.VMEM((PH,), jnp.int32),
            pltpu.VMEM((PH,), jnp.int32),
            pltpu.VMEM((PH, D), F32),
            pltpu.VMEM((PH, D), F32),
            pltpu.VMEM((PH, D), F32),
            pltpu.VMEM((PH, D), F32),
            pltpu.VMEM((n,), F32),
            pltpu.VMEM((n,), F32),
            pltpu.VMEM((n,), F32),
            pltpu.VMEM((PH,), F32),
            pltpu.VMEM((PH,), F32),
            pltpu.SemaphoreType.DMA,
            pltpu.SemaphoreType.DMA,
            pltpu.SemaphoreType.DMA,
            pltpu.SemaphoreType.DMA,
        ],
    )
    def gk(t1_h, t2_h, col_h, row_h, px_h, py_h, pz_h, g1_h, g2_h, d2_h,
           ica_v, ira_v, icb_v, irb_v, r1a_v, r2a_v, r1b_v, r2b_v,
           px_v, py_v, pz_v, d2a_v, d2b_v, si, sa, sb, sw):
        wid = lax.axis_index("s") * 2 + lax.axis_index("c")
        pltpu.sync_copy(px_h, px_v)
        pltpu.sync_copy(py_h, py_v)
        pltpu.sync_copy(pz_h, pz_v)

        def dist(ic_v, ir_v, d2_v):
            for k in range(PH // L):
                ic = ic_v[pl.ds(k * L, L)]
                ir = ir_v[pl.ds(k * L, L)]
                dx = plsc.load_gather(px_v, [ic]) - plsc.load_gather(px_v, [ir])
                dy = plsc.load_gather(py_v, [ic]) - plsc.load_gather(py_v, [ir])
                dz = plsc.load_gather(pz_v, [ic]) - plsc.load_gather(pz_v, [ir])
                d2_v[pl.ds(k * L, L)] = dx * dx + dy * dy + dz * dz

        def fire_gathers(ic_v, ir_v, r1_v, r2_v, sem):
            ds_ = []
            for h in range(PH // CHUNK):
                sl = pl.ds(h * CHUNK, CHUNK)
                ds_.append(pltpu.async_copy(t1_h.at[ic_v.at[sl]], r1_v.at[sl], sem))
                ds_.append(pltpu.async_copy(t2_h.at[ir_v.at[sl]], r2_v.at[sl], sem))
            return ds_

        def body(j, carry):
            base_a = wid * per + j * 2 * PH
            base_b = base_a + PH
            ia1 = pltpu.async_copy(col_h.at[pl.ds(base_a, PH)], ica_v, si)
            ia2 = pltpu.async_copy(row_h.at[pl.ds(base_a, PH)], ira_v, si)
            ib1 = pltpu.async_copy(col_h.at[pl.ds(base_b, PH)], icb_v, si)
            ib2 = pltpu.async_copy(row_h.at[pl.ds(base_b, PH)], irb_v, si)
            ia1.wait()
            ia2.wait()
            ga = fire_gathers(ica_v, ira_v, r1a_v, r2a_v, sa)
            ib1.wait()
            ib2.wait()
            gb = fire_gathers(icb_v, irb_v, r1b_v, r2b_v, sb)
            dist(ica_v, ira_v, d2a_v)
            for dsc in ga:
                dsc.wait()
            wa = [pltpu.async_copy(r1a_v, g1_h.at[pl.ds(base_a, PH)], sw),
                  pltpu.async_copy(r2a_v, g2_h.at[pl.ds(base_a, PH)], sw),
                  pltpu.async_copy(d2a_v, d2_h.at[pl.ds(base_a, PH)], sw)]
            dist(icb_v, irb_v, d2b_v)
            for dsc in gb:
                dsc.wait()
            wb = [pltpu.async_copy(r1b_v, g1_h.at[pl.ds(base_b, PH)], sw),
                  pltpu.async_copy(r2b_v, g2_h.at[pl.ds(base_b, PH)], sw),
                  pltpu.async_copy(d2b_v, d2_h.at[pl.ds(base_b, PH)], sw)]
            for dsc in wa + wb:
                dsc.wait()
            return carry

        lax.fori_loop(0, pairs, body, 0)

    return gk(t1, t2, colp, rowp, px, py, pz)


def _edge_cov_body(g1_ref, g2_ref, d2_ref, at_ref, wd_ref, wa_ref, bias_ref,
                   r_ref):
    d2 = d2_ref[...]
    dist = jnp.sqrt(d2 + 1e-12)
    dp = jnp.clip(dist, 1e-2, 50.0)
    cent = lax.broadcasted_iota(jnp.int32, (1, 64), 1).astype(F32) * (10.0 / 63.0)
    t = dp - cent
    rbf = jnp.exp(-10.0 * t * t)
    h = (g1_ref[...] + g2_ref[...]
         + jnp.dot(rbf, wd_ref[...], preferred_element_type=F32)
         + jnp.dot(at_ref[...], wa_ref[...], preferred_element_type=F32)
         + bias_ref[...])
    r_ref[...] = jnp.maximum(h, 0.0)


def _edge_cov(g1, g2, d2, attr, wd, wattr, biasc, ep):
    eb = 512
    return pl.pallas_call(
        _edge_cov_body,
        grid=(ep // eb,),
        in_specs=[
            pl.BlockSpec((eb, D), lambda i: (i, 0)),
            pl.BlockSpec((eb, D), lambda i: (i, 0)),
            pl.BlockSpec((eb, 1), lambda i: (i, 0)),
            pl.BlockSpec((eb, 16), lambda i: (i, 0)),
            pl.BlockSpec((64, D), lambda i: (0, 0)),
            pl.BlockSpec((16, D), lambda i: (0, 0)),
            pl.BlockSpec((1, D), lambda i: (0, 0)),
        ],
        out_specs=pl.BlockSpec((eb, D), lambda i: (i, 0)),
        out_shape=jax.ShapeDtypeStruct((ep, D), F32),
    )(g1, g2, d2, attr, wd, wattr, biasc)


def _edge_ncov_body(g1_ref, g2_ref, d2_ref, wn_ref, bias_ref, r_ref):
    d2 = d2_ref[...]
    dist = jnp.sqrt(d2 + 1e-12)
    dc = jnp.clip(dist, 1e-2, 50.0)
    dp2 = 1.0 / (dc * dc)
    dp6 = dp2 * dp2 * dp2
    cent = lax.broadcasted_iota(jnp.int32, (1, 64), 1).astype(F32) * (10.0 / 63.0)
    t2 = dp2 - cent
    t6 = dp6 - cent
    rbf = jnp.concatenate([jnp.exp(-10.0 * t2 * t2),
                           jnp.exp(-10.0 * t6 * t6)], axis=1)
    h = (g1_ref[...] + g2_ref[...]
         + jnp.dot(rbf, wn_ref[...], preferred_element_type=F32)
         + bias_ref[...])
    r_ref[...] = jnp.maximum(h, 0.0)


def _edge_ncov(g1, g2, d2, wn, biasn, ep):
    eb = 512
    return pl.pallas_call(
        _edge_ncov_body,
        grid=(ep // eb,),
        in_specs=[
            pl.BlockSpec((eb, D), lambda i: (i, 0)),
            pl.BlockSpec((eb, D), lambda i: (i, 0)),
            pl.BlockSpec((eb, 1), lambda i: (i, 0)),
            pl.BlockSpec((D, D), lambda i: (0, 0)),
            pl.BlockSpec((1, D), lambda i: (0, 0)),
        ],
        out_specs=pl.BlockSpec((eb, D), lambda i: (i, 0)),
        out_shape=jax.ShapeDtypeStruct((ep, D), F32),
    )(g1, g2, d2, wn, biasn)


def _scatter(r, dst, ep):
    per = ep // NW
    chunks = per // CHUNK
    zr = NP // 16

    @functools.partial(
        pl.kernel,
        out_type=(jax.ShapeDtypeStruct((2, NP, D), F32),
                  jax.ShapeDtypeStruct((2, NP), F32)),
        mesh=_sc_mesh(),
        compiler_params=pltpu.CompilerParams(needs_layout_passes=False),
        scratch_types=[
            pltpu.VMEM((CHUNK,), jnp.int32),
            pltpu.VMEM((CHUNK, D), F32),
            pltpu.VMEM((NP,), F32),
            pltpu.VMEM((16, zr), F32),
            pltpu.VMEM((zr,), F32),
            pltpu.VMEM_SHARED((NP, D), F32),
            pltpu.VMEM_SHARED((16, NP), F32),
        ],
    )
    def sk(r_h, dst_h, zm_h, m_h, cnt_h,
           idx_v, rows_v, cnt_v, cbuf_v, cz_v, accm, cstage):
        cid = lax.axis_index("c")
        sid = lax.axis_index("s")
        wid = sid * 2 + cid
        pltpu.sync_copy(zm_h, accm.at[pl.ds(sid * zr, zr)])

        def zbody(i, carry):
            cnt_v[pl.ds(i * L, L)] = jnp.zeros((L,), F32)
            return carry

        lax.fori_loop(0, NP // L, zbody, 0)
        plsc.subcore_barrier()

        def body(j, carry):
            base = wid * per + j * CHUNK
            pltpu.sync_copy(dst_h.at[pl.ds(base, CHUNK)], idx_v)
            pltpu.sync_copy(r_h.at[pl.ds(base, CHUNK)], rows_v)
            pltpu.sync_copy(rows_v, accm.at[idx_v], add=True)
            for k in range(CHUNK // L):
                idx = idx_v[pl.ds(k * L, L)]
                plsc.addupdate_scatter(cnt_v, [idx], jnp.ones((L,), F32))
            return carry

        lax.fori_loop(0, chunks, body, 0)
        pltpu.sync_copy(cnt_v, cstage.at[sid])
        plsc.subcore_barrier()
        pltpu.sync_copy(accm.at[pl.ds(sid * zr, zr)],
                        m_h.at[cid, pl.ds(sid * zr, zr)])
        pltpu.sync_copy(cstage.at[:, pl.ds(sid * zr, zr)], cbuf_v)

        def rbody(g, carry):
            acc = jnp.zeros((L,), F32)
            for rr in range(16):
                acc = acc + cbuf_v[rr, pl.ds(g * L, L)]
            cz_v[pl.ds(g * L, L)] = acc
            return carry

        lax.fori_loop(0, zr // L, rbody, 0)
        pltpu.sync_copy(cz_v, cnt_h.at[cid, pl.ds(sid * zr, zr)])

    zm = jnp.zeros((zr, D), F32)
    return sk(r, dst, zm)


def _final_body(x_ref, mc_ref, cc0_ref, cc1_ref, mn_ref, cn0_ref, cn1_ref,
                wc2_ref, bc2_ref, wn2_ref, bn2_ref,
                wuc_ref, buc_ref, wun_ref, bun_ref, o_ref):
    xb = x_ref[...]
    sc = mc_ref[0] + mc_ref[1]
    cntc = cc0_ref[...] + cc1_ref[...]
    m_cov = (jnp.dot(sc, wc2_ref[...], preferred_element_type=F32)
             + cntc * bc2_ref[...])
    sn = mn_ref[0] + mn_ref[1]
    cntn = cn0_ref[...] + cn1_ref[...]
    m_ncov = (jnp.dot(sn, wn2_ref[...], preferred_element_type=F32)
              + cntn * bn2_ref[...])
    h_cov = jnp.maximum(
        jnp.dot(xb + m_cov, wuc_ref[...], preferred_element_type=F32)
        + buc_ref[...], 0.0)
    h_ncov = jnp.maximum(
        jnp.dot(xb + m_ncov, wun_ref[...], preferred_element_type=F32)
        + bun_ref[...], 0.0)
    o_ref[...] = h_cov + h_ncov


def _final(x, mc, cc0, cc1, mn, cn0, cn1,
           wc2, bc2r, wn2, bn2r, wuc, bucr, wun, bunr):
    n = x.shape[0]
    rb = 1000
    return pl.pallas_call(
        _final_body,
        grid=(n // rb,),
        in_specs=[
            pl.BlockSpec((rb, D), lambda i: (i, 0)),
            pl.BlockSpec((2, rb, D), lambda i: (0, i, 0)),
            pl.BlockSpec((rb, 1), lambda i: (i, 0)),
            pl.BlockSpec((rb, 1), lambda i: (i, 0)),
            pl.BlockSpec((2, rb, D), lambda i: (0, i, 0)),
            pl.BlockSpec((rb, 1), lambda i: (i, 0)),
            pl.BlockSpec((rb, 1), lambda i: (i, 0)),
            pl.BlockSpec((D, D), lambda i: (0, 0)),
            pl.BlockSpec((1, D), lambda i: (0, 0)),
            pl.BlockSpec((D, D), lambda i: (0, 0)),
            pl.BlockSpec((1, D), lambda i: (0, 0)),
            pl.BlockSpec((D, D), lambda i: (0, 0)),
            pl.BlockSpec((1, D), lambda i: (0, 0)),
            pl.BlockSpec((D, D), lambda i: (0, 0)),
            pl.BlockSpec((1, D), lambda i: (0, 0)),
        ],
        out_specs=pl.BlockSpec((rb, D), lambda i: (i, 0)),
        out_shape=jax.ShapeDtypeStruct((n, D), F32),
    )(x, mc, cc0, cc1, mn, cn0, cn1, wc2, bc2r, wn2, bn2r, wuc, bucr, wun, bunr)


def _pad_idx(idx, ep, fill):
    e = idx.shape[0]
    return jnp.concatenate([idx, jnp.full((ep - e,), fill, jnp.int32)])


def kernel(x, pos, edge_index_cov, edge_attr_cov, edge_index_ncov,
           W_bp, b_bp, Wc1, bc1, Wc2, bc2, Wn1, bn1, Wn2, bn2,
           Wuc, buc, Wun, bun):
    ec = edge_index_cov.shape[1]
    en = edge_index_ncov.shape[1]
    epc = -(-ec // (NW * CHUNK)) * (NW * CHUNK)
    epn = -(-en // (NW * CHUNK)) * (NW * CHUNK)

    wcat = jnp.concatenate([Wc1[0:128], Wc1[128:256], Wn1[0:128], Wn1[128:256]],
                           axis=1)
    t1c, t2c, t1n, t2n, wattr, biasc = _prep(
        x, wcat, W_bp, Wc1[256:384], b_bp[None, :], bc1[None, :])

    px, py, pz = pos[:, 0], pos[:, 1], pos[:, 2]
    rowc = _pad_idx(edge_index_cov[0], epc, 0)
    colc = _pad_idx(edge_index_cov[1], epc, 0)
    dstc = _pad_idx(edge_index_cov[1], epc, DUMMY)
    rown = _pad_idx(edge_index_ncov[0], epn, 0)
    coln = _pad_idx(edge_index_ncov[1], epn, 0)
    dstn = _pad_idx(edge_index_ncov[1], epn, DUMMY)
    attrp = jnp.pad(edge_attr_cov, ((0, epc - ec), (0, 0)))

    gc1, gc2, d2c = _gather(t1c, t2c, colc, rowc, px, py, pz, epc)
    gn1, gn2, d2n = _gather(t1n, t2n, coln, rown, px, py, pz, epn)

    rc = _edge_cov(gc1, gc2, d2c[:, None], attrp, Wc1[384:448], wattr, biasc,
                   epc)
    rn = _edge_ncov(gn1, gn2, d2n[:, None], Wn1[256:384], bn1[None, :], epn)

    mc, cc = _scatter(rc, dstc, epc)
    mn, cn = _scatter(rn, dstn, epn)

    out = _final(x, mc, cc[0][:, None], cc[1][:, None],
                 mn, cn[0][:, None], cn[1][:, None],
                 Wc2, bc2[None, :], Wn2, bn2[None, :],
                 Wuc, buc[None, :], Wun, bun[None, :])
    return out


# speedup vs baseline: 2.7391x; 1.0184x over previous
import functools

import jax
import jax.numpy as jnp
from jax import lax
from jax.experimental import pallas as pl
from jax.experimental.pallas import tpu as pltpu
from jax.experimental.pallas import tpu_sc as plsc

F32 = jnp.float32
BF16 = jnp.bfloat16
D = 128
NW = 32
L = 16
CHUNK = 128
NP = 10240
DUMMY = 10100


def _sc_mesh():
    return plsc.VectorSubcoreMesh(core_axis_name="c", subcore_axis_name="s",
                                  num_cores=2, num_subcores=16)


def _prep_body(x_ref, wcat_ref, wbp_ref, wc1c_ref, bbp_ref, bc1_ref,
               t1c_ref, t2c_ref, t1n_ref, t2n_ref, wattr_ref, biasc_ref):
    P = jnp.dot(x_ref[...], wcat_ref[...], preferred_element_type=F32)
    t1c_ref[...] = P[:, 0:128]
    t2c_ref[...] = P[:, 128:256]
    t1n_ref[...] = P[:, 256:384]
    t2n_ref[...] = P[:, 384:512]

    @pl.when(pl.program_id(0) == 0)
    def _():
        wattr_ref[...] = jnp.dot(wbp_ref[...], wc1c_ref[...],
                                 preferred_element_type=F32)
        biasc_ref[...] = jnp.dot(bbp_ref[...], wc1c_ref[...],
                                 preferred_element_type=F32) + bc1_ref[...]


def _prep(x, wcat, wbp, wc1c, bbp_row, bc1_row):
    n = x.shape[0]
    rb = 1000
    return pl.pallas_call(
        _prep_body,
        grid=(n // rb,),
        in_specs=[
            pl.BlockSpec((rb, D), lambda i: (i, 0)),
            pl.BlockSpec((D, 512), lambda i: (0, 0)),
            pl.BlockSpec((16, D), lambda i: (0, 0)),
            pl.BlockSpec((D, D), lambda i: (0, 0)),
            pl.BlockSpec((1, D), lambda i: (0, 0)),
            pl.BlockSpec((1, D), lambda i: (0, 0)),
        ],
        out_specs=[
            pl.BlockSpec((rb, D), lambda i: (i, 0)),
            pl.BlockSpec((rb, D), lambda i: (i, 0)),
            pl.BlockSpec((rb, D), lambda i: (i, 0)),
            pl.BlockSpec((rb, D), lambda i: (i, 0)),
            pl.BlockSpec((16, D), lambda i: (0, 0)),
            pl.BlockSpec((1, D), lambda i: (0, 0)),
        ],
        out_shape=[
            jax.ShapeDtypeStruct((n, D), F32),
            jax.ShapeDtypeStruct((n, D), F32),
            jax.ShapeDtypeStruct((n, D), F32),
            jax.ShapeDtypeStruct((n, D), F32),
            jax.ShapeDtypeStruct((16, D), F32),
            jax.ShapeDtypeStruct((1, D), F32),
        ],
    )(x, wcat, wbp, wc1c, bbp_row, bc1_row)


def _gather(t1, t2, colp, rowp, px, py, pz, ep):
    per = ep // NW
    chunks = per // CHUNK
    n = px.shape[0]

    @functools.partial(
        pl.kernel,
        out_type=(jax.ShapeDtypeStruct((ep, D), F32),
                  jax.ShapeDtypeStruct((ep, D), F32),
                  jax.ShapeDtypeStruct((ep,), F32)),
        mesh=_sc_mesh(),
        compiler_params=pltpu.CompilerParams(needs_layout_passes=False),
        scratch_types=[
            pltpu.VMEM((CHUNK,), jnp.int32),
            pltpu.VMEM((CHUNK,), jnp.int32),
            pltpu.VMEM((CHUNK, D), F32),
            pltpu.VMEM((CHUNK, D), F32),
            pltpu.VMEM((n,), F32),
            pltpu.VMEM((n,), F32),
            pltpu.VMEM((n,), F32),
            pltpu.VMEM((CHUNK,), F32),
            pltpu.SemaphoreType.DMA,
            pltpu.SemaphoreType.DMA,
        ],
    )
    def gk(t1_h, t2_h, col_h, row_h, px_h, py_h, pz_h, g1_h, g2_h, d2_h,
           ic_v, ir_v, r1_v, r2_v, px_v, py_v, pz_v, d2_v, s1, s2):
        wid = lax.axis_index("s") * 2 + lax.axis_index("c")
        pltpu.sync_copy(px_h, px_v)
        pltpu.sync_copy(py_h, py_v)
        pltpu.sync_copy(pz_h, pz_v)

        def body(j, carry):
            base = wid * per + j * CHUNK
            i1 = pltpu.async_copy(col_h.at[pl.ds(base, CHUNK)], ic_v, s1)
            i2 = pltpu.async_copy(row_h.at[pl.ds(base, CHUNK)], ir_v, s1)
            i1.wait()
            i2.wait()
            c1 = pltpu.async_copy(t1_h.at[ic_v], r1_v, s1)
            c2 = pltpu.async_copy(t2_h.at[ir_v], r2_v, s2)
            for k in range(CHUNK // L):
                ic = ic_v[pl.ds(k * L, L)]
                ir = ir_v[pl.ds(k * L, L)]
                dx = plsc.load_gather(px_v, [ic]) - plsc.load_gather(px_v, [ir])
                dy = plsc.load_gather(py_v, [ic]) - plsc.load_gather(py_v, [ir])
                dz = plsc.load_gather(pz_v, [ic]) - plsc.load_gather(pz_v, [ir])
                d2_v[pl.ds(k * L, L)] = dx * dx + dy * dy + dz * dz
            c1.wait()
            c2.wait()
            w1 = pltpu.async_copy(r1_v, g1_h.at[pl.ds(base, CHUNK)], s1)
            w2 = pltpu.async_copy(r2_v, g2_h.at[pl.ds(base, CHUNK)], s2)
            w3 = pltpu.async_copy(d2_v, d2_h.at[pl.ds(base, CHUNK)], s1)
            w1.wait()
            w2.wait()
            w3.wait()
            return carry

        lax.fori_loop(0, chunks, body, 0)

    return gk(t1, t2, colp, rowp, px, py, pz)


def _edge_cov_body(g1_ref, g2_ref, d2_ref, at_ref, wd_ref, wa_ref, bias_ref,
                   r_ref):
    d2 = d2_ref[...]
    dist = jnp.sqrt(d2 + 1e-12)
    dp = jnp.clip(dist, 1e-2, 50.0)
    cent = lax.broadcasted_iota(jnp.int32, (1, 64), 1).astype(F32) * (10.0 / 63.0)
    t = dp - cent
    rbf = jnp.exp(-10.0 * t * t)
    h = (g1_ref[...] + g2_ref[...]
         + jnp.dot(rbf, wd_ref[...], preferred_element_type=F32)
         + jnp.dot(at_ref[...], wa_ref[...], preferred_element_type=F32)
         + bias_ref[...])
    r_ref[...] = jnp.maximum(h, 0.0)


def _edge_cov(g1, g2, d2, attr, wd, wattr, biasc, ep):
    eb = 512
    return pl.pallas_call(
        _edge_cov_body,
        grid=(ep // eb,),
        in_specs=[
            pl.BlockSpec((eb, D), lambda i: (i, 0)),
            pl.BlockSpec((eb, D), lambda i: (i, 0)),
            pl.BlockSpec((eb, 1), lambda i: (i, 0)),
            pl.BlockSpec((eb, 16), lambda i: (i, 0)),
            pl.BlockSpec((64, D), lambda i: (0, 0)),
            pl.BlockSpec((16, D), lambda i: (0, 0)),
            pl.BlockSpec((1, D), lambda i: (0, 0)),
        ],
        out_specs=pl.BlockSpec((eb, D), lambda i: (i, 0)),
        out_shape=jax.ShapeDtypeStruct((ep, D), F32),
    )(g1, g2, d2, attr, wd, wattr, biasc)


def _edge_ncov_body(g1_ref, g2_ref, d2_ref, wn_ref, bias_ref, r_ref):
    d2 = d2_ref[...]
    dist = jnp.sqrt(d2 + 1e-12)
    dc = jnp.clip(dist, 1e-2, 50.0)
    dp2 = 1.0 / (dc * dc)
    dp6 = dp2 * dp2 * dp2
    cent = lax.broadcasted_iota(jnp.int32, (1, 64), 1).astype(F32) * (10.0 / 63.0)
    t2 = dp2 - cent
    t6 = dp6 - cent
    rbf = jnp.concatenate([jnp.exp(-10.0 * t2 * t2),
                           jnp.exp(-10.0 * t6 * t6)], axis=1)
    h = (g1_ref[...] + g2_ref[...]
         + jnp.dot(rbf, wn_ref[...], preferred_element_type=F32)
         + bias_ref[...])
    r_ref[...] = jnp.maximum(h, 0.0)


def _edge_ncov(g1, g2, d2, wn, biasn, ep):
    eb = 512
    return pl.pallas_call(
        _edge_ncov_body,
        grid=(ep // eb,),
        in_specs=[
            pl.BlockSpec((eb, D), lambda i: (i, 0)),
            pl.BlockSpec((eb, D), lambda i: (i, 0)),
            pl.BlockSpec((eb, 1), lambda i: (i, 0)),
            pl.BlockSpec((D, D), lambda i: (0, 0)),
            pl.BlockSpec((1, D), lambda i: (0, 0)),
        ],
        out_specs=pl.BlockSpec((eb, D), lambda i: (i, 0)),
        out_shape=jax.ShapeDtypeStruct((ep, D), F32),
    )(g1, g2, d2, wn, biasn)


def _scatter(r, dst, ep):
    per = ep // NW
    chunks = per // CHUNK
    zr = NP // 16

    @functools.partial(
        pl.kernel,
        out_type=(jax.ShapeDtypeStruct((2, NP, D), F32),
                  jax.ShapeDtypeStruct((2, NP), F32)),
        mesh=_sc_mesh(),
        compiler_params=pltpu.CompilerParams(needs_layout_passes=False),
        scratch_types=[
            pltpu.VMEM((CHUNK,), jnp.int32),
            pltpu.VMEM((CHUNK, D), F32),
            pltpu.VMEM((NP,), F32),
            pltpu.VMEM((16, zr), F32),
            pltpu.VMEM((zr,), F32),
            pltpu.VMEM_SHARED((NP, D), F32),
            pltpu.VMEM_SHARED((16, NP), F32),
        ],
    )
    def sk(r_h, dst_h, zm_h, m_h, cnt_h,
           idx_v, rows_v, cnt_v, cbuf_v, cz_v, accm, cstage):
        cid = lax.axis_index("c")
        sid = lax.axis_index("s")
        wid = sid * 2 + cid
        pltpu.sync_copy(zm_h, accm.at[pl.ds(sid * zr, zr)])

        def zbody(i, carry):
            cnt_v[pl.ds(i * L, L)] = jnp.zeros((L,), F32)
            return carry

        lax.fori_loop(0, NP // L, zbody, 0)
        plsc.subcore_barrier()

        def body(j, carry):
            base = wid * per + j * CHUNK
            pltpu.sync_copy(dst_h.at[pl.ds(base, CHUNK)], idx_v)
            pltpu.sync_copy(r_h.at[pl.ds(base, CHUNK)], rows_v)
            pltpu.sync_copy(rows_v, accm.at[idx_v], add=True)
            for k in range(CHUNK // L):
                idx = idx_v[pl.ds(k * L, L)]
                plsc.addupdate_scatter(cnt_v, [idx], jnp.ones((L,), F32))
            return carry

        lax.fori_loop(0, chunks, body, 0)
        pltpu.sync_copy(cnt_v, cstage.at[sid])
        plsc.subcore_barrier()
        pltpu.sync_copy(accm.at[pl.ds(sid * zr, zr)],
                        m_h.at[cid, pl.ds(sid * zr, zr)])
        pltpu.sync_copy(cstage.at[:, pl.ds(sid * zr, zr)], cbuf_v)

        def rbody(g, carry):
            acc = jnp.zeros((L,), F32)
            for rr in range(16):
                acc = acc + cbuf_v[rr, pl.ds(g * L, L)]
            cz_v[pl.ds(g * L, L)] = acc
            return carry

        lax.fori_loop(0, zr // L, rbody, 0)
        pltpu.sync_copy(cz_v, cnt_h.at[cid, pl.ds(sid * zr, zr)])

    zm = jnp.zeros((zr, D), F32)
    return sk(r, dst, zm)


def _final_body(x_ref, mc_ref, cc0_ref, cc1_ref, mn_ref, cn0_ref, cn1_ref,
                wc2_ref, bc2_ref, wn2_ref, bn2_ref,
                wuc_ref, buc_ref, wun_ref, bun_ref, o_ref):
    xb = x_ref[...]
    sc = mc_ref[0] + mc_ref[1]
    cntc = cc0_ref[...] + cc1_ref[...]
    m_cov = (jnp.dot(sc, wc2_ref[...], preferred_element_type=F32)
             + cntc * bc2_ref[...])
    sn = mn_ref[0] + mn_ref[1]
    cntn = cn0_ref[...] + cn1_ref[...]
    m_ncov = (jnp.dot(sn, wn2_ref[...], preferred_element_type=F32)
              + cntn * bn2_ref[...])
    h_cov = jnp.maximum(
        jnp.dot(xb + m_cov, wuc_ref[...], preferred_element_type=F32)
        + buc_ref[...], 0.0)
    h_ncov = jnp.maximum(
        jnp.dot(xb + m_ncov, wun_ref[...], preferred_element_type=F32)
        + bun_ref[...], 0.0)
    o_ref[...] = h_cov + h_ncov


def _final(x, mc, cc0, cc1, mn, cn0, cn1,
           wc2, bc2r, wn2, bn2r, wuc, bucr, wun, bunr):
    n = x.shape[0]
    rb = 1000
    return pl.pallas_call(
        _final_body,
        grid=(n // rb,),
        in_specs=[
            pl.BlockSpec((rb, D), lambda i: (i, 0)),
            pl.BlockSpec((2, rb, D), lambda i: (0, i, 0)),
            pl.BlockSpec((rb, 1), lambda i: (i, 0)),
            pl.BlockSpec((rb, 1), lambda i: (i, 0)),
            pl.BlockSpec((2, rb, D), lambda i: (0, i, 0)),
            pl.BlockSpec((rb, 1), lambda i: (i, 0)),
            pl.BlockSpec((rb, 1), lambda i: (i, 0)),
            pl.BlockSpec((D, D), lambda i: (0, 0)),
            pl.BlockSpec((1, D), lambda i: (0, 0)),
            pl.BlockSpec((D, D), lambda i: (0, 0)),
            pl.BlockSpec((1, D), lambda i: (0, 0)),
            pl.BlockSpec((D, D), lambda i: (0, 0)),
            pl.BlockSpec((1, D), lambda i: (0, 0)),
            pl.BlockSpec((D, D), lambda i: (0, 0)),
            pl.BlockSpec((1, D), lambda i: (0, 0)),
        ],
        out_specs=pl.BlockSpec((rb, D), lambda i: (i, 0)),
        out_shape=jax.ShapeDtypeStruct((n, D), F32),
    )(x, mc, cc0, cc1, mn, cn0, cn1, wc2, bc2r, wn2, bn2r, wuc, bucr, wun, bunr)


def _pad_idx(idx, ep, fill):
    e = idx.shape[0]
    return jnp.concatenate([idx, jnp.full((ep - e,), fill, jnp.int32)])


def kernel(x, pos, edge_index_cov, edge_attr_cov, edge_index_ncov,
           W_bp, b_bp, Wc1, bc1, Wc2, bc2, Wn1, bn1, Wn2, bn2,
           Wuc, buc, Wun, bun):
    ec = edge_index_cov.shape[1]
    en = edge_index_ncov.shape[1]
    epc = -(-ec // (NW * CHUNK)) * (NW * CHUNK)
    epn = -(-en // (NW * CHUNK)) * (NW * CHUNK)

    wcat = jnp.concatenate([Wc1[0:128], Wc1[128:256], Wn1[0:128], Wn1[128:256]],
                           axis=1)
    t1c, t2c, t1n, t2n, wattr, biasc = _prep(
        x, wcat, W_bp, Wc1[256:384], b_bp[None, :], bc1[None, :])

    px, py, pz = pos[:, 0], pos[:, 1], pos[:, 2]
    rowc = _pad_idx(edge_index_cov[0], epc, 0)
    colc = _pad_idx(edge_index_cov[1], epc, 0)
    dstc = _pad_idx(edge_index_cov[1], epc, DUMMY)
    rown = _pad_idx(edge_index_ncov[0], epn, 0)
    coln = _pad_idx(edge_index_ncov[1], epn, 0)
    dstn = _pad_idx(edge_index_ncov[1], epn, DUMMY)
    attrp = jnp.pad(edge_attr_cov, ((0, epc - ec), (0, 0)))

    gc1, gc2, d2c = _gather(t1c, t2c, colc, rowc, px, py, pz, epc)
    gn1, gn2, d2n = _gather(t1n, t2n, coln, rown, px, py, pz, epn)

    rc = _edge_cov(gc1, gc2, d2c[:, None], attrp, Wc1[384:448], wattr, biasc,
                   epc)
    rn = _edge_ncov(gn1, gn2, d2n[:, None], Wn1[256:384], bn1[None, :], epn)

    mc, cc = _scatter(rc, dstc, epc)
    mn, cn = _scatter(rn, dstn, epn)

    out = _final(x, mc, cc[0][:, None], cc[1][:, None],
                 mn, cn[0][:, None], cn[1][:, None],
                 Wc2, bc2[None, :], Wn2, bn2[None, :],
                 Wuc, buc[None, :], Wun, bun[None, :])
    return out


# speedup vs baseline: 2.7589x; 1.0072x over previous
import functools

import jax
import jax.numpy as jnp
from jax import lax
from jax.experimental import pallas as pl
from jax.experimental.pallas import tpu as pltpu
from jax.experimental.pallas import tpu_sc as plsc

F32 = jnp.float32
BF16 = jnp.bfloat16
D = 128
NW = 32
L = 16
CHUNK = 128
NP = 10240
DUMMY = 10100


def _sc_mesh():
    return plsc.VectorSubcoreMesh(core_axis_name="c", subcore_axis_name="s",
                                  num_cores=2, num_subcores=16)


def _prep_body(x_ref, wcat_ref, wbp_ref, wc1c_ref, bbp_ref, bc1_ref,
               t1c_ref, t2c_ref, t1n_ref, t2n_ref, wattr_ref, biasc_ref):
    P = jnp.dot(x_ref[...], wcat_ref[...], preferred_element_type=F32)
    t1c_ref[...] = P[:, 0:128]
    t2c_ref[...] = P[:, 128:256]
    t1n_ref[...] = P[:, 256:384]
    t2n_ref[...] = P[:, 384:512]

    @pl.when(pl.program_id(0) == 0)
    def _():
        wattr_ref[...] = jnp.dot(wbp_ref[...], wc1c_ref[...],
                                 preferred_element_type=F32)
        biasc_ref[...] = jnp.dot(bbp_ref[...], wc1c_ref[...],
                                 preferred_element_type=F32) + bc1_ref[...]


def _prep(x, wcat, wbp, wc1c, bbp_row, bc1_row):
    n = x.shape[0]
    rb = 1000
    return pl.pallas_call(
        _prep_body,
        grid=(n // rb,),
        in_specs=[
            pl.BlockSpec((rb, D), lambda i: (i, 0)),
            pl.BlockSpec((D, 512), lambda i: (0, 0)),
            pl.BlockSpec((16, D), lambda i: (0, 0)),
            pl.BlockSpec((D, D), lambda i: (0, 0)),
            pl.BlockSpec((1, D), lambda i: (0, 0)),
            pl.BlockSpec((1, D), lambda i: (0, 0)),
        ],
        out_specs=[
            pl.BlockSpec((rb, D), lambda i: (i, 0)),
            pl.BlockSpec((rb, D), lambda i: (i, 0)),
            pl.BlockSpec((rb, D), lambda i: (i, 0)),
            pl.BlockSpec((rb, D), lambda i: (i, 0)),
            pl.BlockSpec((16, D), lambda i: (0, 0)),
            pl.BlockSpec((1, D), lambda i: (0, 0)),
        ],
        out_shape=[
            jax.ShapeDtypeStruct((n, D), F32),
            jax.ShapeDtypeStruct((n, D), F32),
            jax.ShapeDtypeStruct((n, D), F32),
            jax.ShapeDtypeStruct((n, D), F32),
            jax.ShapeDtypeStruct((16, D), F32),
            jax.ShapeDtypeStruct((1, D), F32),
        ],
    )(x, wcat, wbp, wc1c, bbp_row, bc1_row)


def _gather(t1, t2, colp, rowp, px, py, pz, ep):
    per = ep // NW
    chunks = per // CHUNK
    n = px.shape[0]

    @functools.partial(
        pl.kernel,
        out_type=(jax.ShapeDtypeStruct((ep, D), F32),
                  jax.ShapeDtypeStruct((ep, D), F32),
                  jax.ShapeDtypeStruct((ep,), F32)),
        mesh=_sc_mesh(),
        compiler_params=pltpu.CompilerParams(needs_layout_passes=False),
        scratch_types=[
            pltpu.VMEM((CHUNK,), jnp.int32),
            pltpu.VMEM((CHUNK,), jnp.int32),
            pltpu.VMEM((CHUNK, D), F32),
            pltpu.VMEM((CHUNK, D), F32),
            pltpu.VMEM((n,), F32),
            pltpu.VMEM((n,), F32),
            pltpu.VMEM((n,), F32),
            pltpu.VMEM((CHUNK,), F32),
            pltpu.SemaphoreType.DMA,
            pltpu.SemaphoreType.DMA,
        ],
    )
    def gk(t1_h, t2_h, col_h, row_h, px_h, py_h, pz_h, g1_h, g2_h, d2_h,
           ic_v, ir_v, r1_v, r2_v, px_v, py_v, pz_v, d2_v, s1, s2):
        wid = lax.axis_index("s") * 2 + lax.axis_index("c")
        pltpu.sync_copy(px_h, px_v)
        pltpu.sync_copy(py_h, py_v)
        pltpu.sync_copy(pz_h, pz_v)

        def body(j, carry):
            base = wid * per + j * CHUNK
            i1 = pltpu.async_copy(col_h.at[pl.ds(base, CHUNK)], ic_v, s1)
            i2 = pltpu.async_copy(row_h.at[pl.ds(base, CHUNK)], ir_v, s1)
            i1.wait()
            i2.wait()
            c1 = pltpu.async_copy(t1_h.at[ic_v], r1_v, s1)
            c2 = pltpu.async_copy(t2_h.at[ir_v], r2_v, s2)
            for k in range(CHUNK // L):
                ic = ic_v[pl.ds(k * L, L)]
                ir = ir_v[pl.ds(k * L, L)]
                dx = plsc.load_gather(px_v, [ic]) - plsc.load_gather(px_v, [ir])
                dy = plsc.load_gather(py_v, [ic]) - plsc.load_gather(py_v, [ir])
                dz = plsc.load_gather(pz_v, [ic]) - plsc.load_gather(pz_v, [ir])
                d2_v[pl.ds(k * L, L)] = dx * dx + dy * dy + dz * dz
            c1.wait()
            c2.wait()
            w1 = pltpu.async_copy(r1_v, g1_h.at[pl.ds(base, CHUNK)], s1)
            w2 = pltpu.async_copy(r2_v, g2_h.at[pl.ds(base, CHUNK)], s2)
            w3 = pltpu.async_copy(d2_v, d2_h.at[pl.ds(base, CHUNK)], s1)
            w1.wait()
            w2.wait()
            w3.wait()
            return carry

        lax.fori_loop(0, chunks, body, 0)

    return gk(t1, t2, colp, rowp, px, py, pz)


def _edge_cov_body(g1_ref, g2_ref, d2_ref, at_ref, wd_ref, wa_ref, bias_ref,
                   r_ref):
    d2 = d2_ref[...]
    dist = jnp.sqrt(d2 + 1e-12)
    dp = jnp.clip(dist, 1e-2, 50.0)
    cent = lax.broadcasted_iota(jnp.int32, (1, 64), 1).astype(F32) * (10.0 / 63.0)
    t = dp - cent
    rbf = jnp.exp(-10.0 * t * t)
    h = (g1_ref[...] + g2_ref[...]
         + jnp.dot(rbf, wd_ref[...], preferred_element_type=F32)
         + jnp.dot(at_ref[...], wa_ref[...], preferred_element_type=F32)
         + bias_ref[...])
    r_ref[...] = jnp.maximum(h, 0.0)


def _edge_cov(g1, g2, d2, attr, wd, wattr, biasc, ep):
    eb = 512
    return pl.pallas_call(
        _edge_cov_body,
        grid=(ep // eb,),
        in_specs=[
            pl.BlockSpec((eb, D), lambda i: (i, 0)),
            pl.BlockSpec((eb, D), lambda i: (i, 0)),
            pl.BlockSpec((eb, 1), lambda i: (i, 0)),
            pl.BlockSpec((eb, 16), lambda i: (i, 0)),
            pl.BlockSpec((64, D), lambda i: (0, 0)),
            pl.BlockSpec((16, D), lambda i: (0, 0)),
            pl.BlockSpec((1, D), lambda i: (0, 0)),
        ],
        out_specs=pl.BlockSpec((eb, D), lambda i: (i, 0)),
        out_shape=jax.ShapeDtypeStruct((ep, D), F32),
    )(g1, g2, d2, attr, wd, wattr, biasc)


def _edge_ncov_body(g1_ref, g2_ref, d2_ref, wn_ref, bias_ref, r_ref):
    d2 = d2_ref[...]
    dist = jnp.sqrt(d2 + 1e-12)
    dc = jnp.clip(dist, 1e-2, 50.0)
    dp2 = 1.0 / (dc * dc)
    dp6 = dp2 * dp2 * dp2
    cent = lax.broadcasted_iota(jnp.int32, (1, 64), 1).astype(F32) * (10.0 / 63.0)
    t2 = dp2 - cent
    t6 = dp6 - cent
    rbf = jnp.concatenate([jnp.exp(-10.0 * t2 * t2),
                           jnp.exp(-10.0 * t6 * t6)], axis=1)
    h = (g1_ref[...] + g2_ref[...]
         + jnp.dot(rbf, wn_ref[...], preferred_element_type=F32)
         + bias_ref[...])
    r_ref[...] = jnp.maximum(h, 0.0)


def _edge_ncov(g1, g2, d2, wn, biasn, ep):
    eb = 512
    return pl.pallas_call(
        _edge_ncov_body,
        grid=(ep // eb,),
        in_specs=[
            pl.BlockSpec((eb, D), lambda i: (i, 0)),
            pl.BlockSpec((eb, D), lambda i: (i, 0)),
            pl.BlockSpec((eb, 1), lambda i: (i, 0)),
            pl.BlockSpec((D, D), lambda i: (0, 0)),
            pl.BlockSpec((1, D), lambda i: (0, 0)),
        ],
        out_specs=pl.BlockSpec((eb, D), lambda i: (i, 0)),
        out_shape=jax.ShapeDtypeStruct((ep, D), F32),
    )(g1, g2, d2, wn, biasn)


def _scatter(r, dst, ep):
    per = ep // NW
    chunks = per // CHUNK
    zr = NP // 16

    @functools.partial(
        pl.kernel,
        out_type=(jax.ShapeDtypeStruct((2, NP, D), F32),
                  jax.ShapeDtypeStruct((2, NP), F32)),
        mesh=_sc_mesh(),
        compiler_params=pltpu.CompilerParams(needs_layout_passes=False),
        scratch_types=[
            pltpu.VMEM((CHUNK,), jnp.int32),
            pltpu.VMEM((CHUNK, D), F32),
            pltpu.VMEM((NP,), F32),
            pltpu.VMEM((16, zr), F32),
            pltpu.VMEM((zr,), F32),
            pltpu.VMEM_SHARED((NP, D), F32),
            pltpu.VMEM_SHARED((16, NP), F32),
            pltpu.SemaphoreType.DMA,
            pltpu.SemaphoreType.DMA,
        ],
    )
    def sk(r_h, dst_h, zm_h, m_h, cnt_h,
           idx_v, rows_v, cnt_v, cbuf_v, cz_v, accm, cstage, s1, s2):
        cid = lax.axis_index("c")
        sid = lax.axis_index("s")
        wid = sid * 2 + cid
        pltpu.sync_copy(zm_h, accm.at[pl.ds(sid * zr, zr)])

        def zbody(i, carry):
            cnt_v[pl.ds(i * L, L)] = jnp.zeros((L,), F32)
            return carry

        lax.fori_loop(0, NP // L, zbody, 0)
        plsc.subcore_barrier()

        def body(j, carry):
            base = wid * per + j * CHUNK
            l1 = pltpu.async_copy(dst_h.at[pl.ds(base, CHUNK)], idx_v, s1)
            l2 = pltpu.async_copy(r_h.at[pl.ds(base, CHUNK)], rows_v, s2)
            l1.wait()
            l2.wait()
            pltpu.sync_copy(rows_v, accm.at[idx_v], add=True)
            for k in range(CHUNK // L):
                idx = idx_v[pl.ds(k * L, L)]
                plsc.addupdate_scatter(cnt_v, [idx], jnp.ones((L,), F32))
            return carry

        lax.fori_loop(0, chunks, body, 0)
        pltpu.sync_copy(cnt_v, cstage.at[sid])
        plsc.subcore_barrier()
        pltpu.sync_copy(accm.at[pl.ds(sid * zr, zr)],
                        m_h.at[cid, pl.ds(sid * zr, zr)])
        pltpu.sync_copy(cstage.at[:, pl.ds(sid * zr, zr)], cbuf_v)

        def rbody(g, carry):
            acc = jnp.zeros((L,), F32)
            for rr in range(16):
                acc = acc + cbuf_v[rr, pl.ds(g * L, L)]
            cz_v[pl.ds(g * L, L)] = acc
            return carry

        lax.fori_loop(0, zr // L, rbody, 0)
        pltpu.sync_copy(cz_v, cnt_h.at[cid, pl.ds(sid * zr, zr)])

    zm = jnp.zeros((zr, D), F32)
    return sk(r, dst, zm)


def _final_body(x_ref, mc_ref, cc0_ref, cc1_ref, mn_ref, cn0_ref, cn1_ref,
                wc2_ref, bc2_ref, wn2_ref, bn2_ref,
                wuc_ref, buc_ref, wun_ref, bun_ref, o_ref):
    xb = x_ref[...]
    sc = mc_ref[0] + mc_ref[1]
    cntc = cc0_ref[...] + cc1_ref[...]
    m_cov = (jnp.dot(sc, wc2_ref[...], preferred_element_type=F32)
             + cntc * bc2_ref[...])
    sn = mn_ref[0] + mn_ref[1]
    cntn = cn0_ref[...] + cn1_ref[...]
    m_ncov = (jnp.dot(sn, wn2_ref[...], preferred_element_type=F32)
              + cntn * bn2_ref[...])
    h_cov = jnp.maximum(
        jnp.dot(xb + m_cov, wuc_ref[...], preferred_element_type=F32)
        + buc_ref[...], 0.0)
    h_ncov = jnp.maximum(
        jnp.dot(xb + m_ncov, wun_ref[...], preferred_element_type=F32)
        + bun_ref[...], 0.0)
    o_ref[...] = h_cov + h_ncov


def _final(x, mc, cc0, cc1, mn, cn0, cn1,
           wc2, bc2r, wn2, bn2r, wuc, bucr, wun, bunr):
    n = x.shape[0]
    rb = 1000
    return pl.pallas_call(
        _final_body,
        grid=(n // rb,),
        in_specs=[
            pl.BlockSpec((rb, D), lambda i: (i, 0)),
            pl.BlockSpec((2, rb, D), lambda i: (0, i, 0)),
            pl.BlockSpec((rb, 1), lambda i: (i, 0)),
            pl.BlockSpec((rb, 1), lambda i: (i, 0)),
            pl.BlockSpec((2, rb, D), lambda i: (0, i, 0)),
            pl.BlockSpec((rb, 1), lambda i: (i, 0)),
            pl.BlockSpec((rb, 1), lambda i: (i, 0)),
            pl.BlockSpec((D, D), lambda i: (0, 0)),
            pl.BlockSpec((1, D), lambda i: (0, 0)),
            pl.BlockSpec((D, D), lambda i: (0, 0)),
            pl.BlockSpec((1, D), lambda i: (0, 0)),
            pl.BlockSpec((D, D), lambda i: (0, 0)),
            pl.BlockSpec((1, D), lambda i: (0, 0)),
            pl.BlockSpec((D, D), lambda i: (0, 0)),
            pl.BlockSpec((1, D), lambda i: (0, 0)),
        ],
        out_specs=pl.BlockSpec((rb, D), lambda i: (i, 0)),
        out_shape=jax.ShapeDtypeStruct((n, D), F32),
    )(x, mc, cc0, cc1, mn, cn0, cn1, wc2, bc2r, wn2, bn2r, wuc, bucr, wun, bunr)


def _pad_idx(idx, ep, fill):
    e = idx.shape[0]
    return jnp.concatenate([idx, jnp.full((ep - e,), fill, jnp.int32)])


def kernel(x, pos, edge_index_cov, edge_attr_cov, edge_index_ncov,
           W_bp, b_bp, Wc1, bc1, Wc2, bc2, Wn1, bn1, Wn2, bn2,
           Wuc, buc, Wun, bun):
    ec = edge_index_cov.shape[1]
    en = edge_index_ncov.shape[1]
    epc = -(-ec // (NW * CHUNK)) * (NW * CHUNK)
    epn = -(-en // (NW * CHUNK)) * (NW * CHUNK)

    wcat = jnp.concatenate([Wc1[0:128], Wc1[128:256], Wn1[0:128], Wn1[128:256]],
                           axis=1)
    t1c, t2c, t1n, t2n, wattr, biasc = _prep(
        x, wcat, W_bp, Wc1[256:384], b_bp[None, :], bc1[None, :])

    px, py, pz = pos[:, 0], pos[:, 1], pos[:, 2]
    rowc = _pad_idx(edge_index_cov[0], epc, 0)
    colc = _pad_idx(edge_index_cov[1], epc, 0)
    dstc = _pad_idx(edge_index_cov[1], epc, DUMMY)
    rown = _pad_idx(edge_index_ncov[0], epn, 0)
    coln = _pad_idx(edge_index_ncov[1], epn, 0)
    dstn = _pad_idx(edge_index_ncov[1], epn, DUMMY)
    attrp = jnp.pad(edge_attr_cov, ((0, epc - ec), (0, 0)))

    gc1, gc2, d2c = _gather(t1c, t2c, colc, rowc, px, py, pz, epc)
    gn1, gn2, d2n = _gather(t1n, t2n, coln, rown, px, py, pz, epn)

    rc = _edge_cov(gc1, gc2, d2c[:, None], attrp, Wc1[384:448], wattr, biasc,
                   epc)
    rn = _edge_ncov(gn1, gn2, d2n[:, None], Wn1[256:384], bn1[None, :], epn)

    mc, cc = _scatter(rc, dstc, epc)
    mn, cn = _scatter(rn, dstn, epn)

    out = _final(x, mc, cc[0][:, None], cc[1][:, None],
                 mn, cn[0][:, None], cn[1][:, None],
                 Wc2, bc2[None, :], Wn2, bn2[None, :],
                 Wuc, buc[None, :], Wun, bun[None, :])
    return out


# speedup vs baseline: 2.7672x; 1.0030x over previous
import functools

import jax
import jax.numpy as jnp
from jax import lax
from jax.experimental import pallas as pl
from jax.experimental.pallas import tpu as pltpu
from jax.experimental.pallas import tpu_sc as plsc

F32 = jnp.float32
D = 128
NW = 32
L = 16
CHUNK = 128
NP = 10240
DUMMY = 10100


def _sc_mesh():
    return plsc.VectorSubcoreMesh(core_axis_name="c", subcore_axis_name="s",
                                  num_cores=2, num_subcores=16)


def _prep_body(x_ref, wcat_ref, wbp_ref, wc1c_ref, bbp_ref, bc1_ref,
               t1c_ref, t2c_ref, t1n_ref, t2n_ref, wattr_ref, biasc_ref):
    P = jnp.dot(x_ref[...], wcat_ref[...], preferred_element_type=F32)
    t1c_ref[...] = P[:, 0:128]
    t2c_ref[...] = P[:, 128:256]
    t1n_ref[...] = P[:, 256:384]
    t2n_ref[...] = P[:, 384:512]

    @pl.when(pl.program_id(0) == 0)
    def _():
        wattr_ref[...] = jnp.dot(wbp_ref[...], wc1c_ref[...],
                                 preferred_element_type=F32)
        biasc_ref[...] = jnp.dot(bbp_ref[...], wc1c_ref[...],
                                 preferred_element_type=F32) + bc1_ref[...]


def _prep(x, wcat, wbp, wc1c, bbp_row, bc1_row):
    n = x.shape[0]
    rb = 1000
    return pl.pallas_call(
        _prep_body,
        grid=(n // rb,),
        in_specs=[
            pl.BlockSpec((rb, D), lambda i: (i, 0)),
            pl.BlockSpec((D, 512), lambda i: (0, 0)),
            pl.BlockSpec((16, D), lambda i: (0, 0)),
            pl.BlockSpec((D, D), lambda i: (0, 0)),
            pl.BlockSpec((1, D), lambda i: (0, 0)),
            pl.BlockSpec((1, D), lambda i: (0, 0)),
        ],
        out_specs=[
            pl.BlockSpec((rb, D), lambda i: (i, 0)),
            pl.BlockSpec((rb, D), lambda i: (i, 0)),
            pl.BlockSpec((rb, D), lambda i: (i, 0)),
            pl.BlockSpec((rb, D), lambda i: (i, 0)),
            pl.BlockSpec((16, D), lambda i: (0, 0)),
            pl.BlockSpec((1, D), lambda i: (0, 0)),
        ],
        out_shape=[
            jax.ShapeDtypeStruct((n, D), F32),
            jax.ShapeDtypeStruct((n, D), F32),
            jax.ShapeDtypeStruct((n, D), F32),
            jax.ShapeDtypeStruct((n, D), F32),
            jax.ShapeDtypeStruct((16, D), F32),
            jax.ShapeDtypeStruct((1, D), F32),
        ],
    )(x, wcat, wbp, wc1c, bbp_row, bc1_row)


def _gather(t1, t2, colp, rowp, px, py, pz, ep):
    per = ep // NW
    chunks = per // CHUNK
    n = px.shape[0]

    @functools.partial(
        pl.kernel,
        out_type=(jax.ShapeDtypeStruct((ep, D), F32),
                  jax.ShapeDtypeStruct((ep, D), F32),
                  jax.ShapeDtypeStruct((ep,), F32)),
        mesh=_sc_mesh(),
        compiler_params=pltpu.CompilerParams(needs_layout_passes=False),
        scratch_types=[
            pltpu.VMEM((CHUNK,), jnp.int32),
            pltpu.VMEM((CHUNK,), jnp.int32),
            pltpu.VMEM((CHUNK, D), F32),
            pltpu.VMEM((CHUNK, D), F32),
            pltpu.VMEM((n,), F32),
            pltpu.VMEM((n,), F32),
            pltpu.VMEM((n,), F32),
            pltpu.VMEM((CHUNK,), F32),
            pltpu.SemaphoreType.DMA,
            pltpu.SemaphoreType.DMA,
        ],
    )
    def gk(t1_h, t2_h, col_h, row_h, px_h, py_h, pz_h, g1_h, g2_h, d2_h,
           ic_v, ir_v, r1_v, r2_v, px_v, py_v, pz_v, d2_v, s1, s2):
        wid = lax.axis_index("s") * 2 + lax.axis_index("c")
        pltpu.sync_copy(px_h, px_v)
        pltpu.sync_copy(py_h, py_v)
        pltpu.sync_copy(pz_h, pz_v)

        def body(j, carry):
            base = wid * per + j * CHUNK
            i1 = pltpu.async_copy(col_h.at[pl.ds(base, CHUNK)], ic_v, s1)
            i2 = pltpu.async_copy(row_h.at[pl.ds(base, CHUNK)], ir_v, s1)
            i1.wait()
            i2.wait()
            c1 = pltpu.async_copy(t1_h.at[ic_v], r1_v, s1)
            c2 = pltpu.async_copy(t2_h.at[ir_v], r2_v, s2)
            for k in range(CHUNK // L):
                ic = ic_v[pl.ds(k * L, L)]
                ir = ir_v[pl.ds(k * L, L)]
                dx = plsc.load_gather(px_v, [ic]) - plsc.load_gather(px_v, [ir])
                dy = plsc.load_gather(py_v, [ic]) - plsc.load_gather(py_v, [ir])
                dz = plsc.load_gather(pz_v, [ic]) - plsc.load_gather(pz_v, [ir])
                d2_v[pl.ds(k * L, L)] = dx * dx + dy * dy + dz * dz
            c1.wait()
            c2.wait()
            w1 = pltpu.async_copy(r1_v, g1_h.at[pl.ds(base, CHUNK)], s1)
            w2 = pltpu.async_copy(r2_v, g2_h.at[pl.ds(base, CHUNK)], s2)
            w3 = pltpu.async_copy(d2_v, d2_h.at[pl.ds(base, CHUNK)], s1)
            w1.wait()
            w2.wait()
            w3.wait()
            return carry

        lax.fori_loop(0, chunks, body, 0)

    return gk(t1, t2, colp, rowp, px, py, pz)


def _edge_cov_body(g1_ref, g2_ref, d2_ref, at_ref, wd_ref, wa_ref, bias_ref,
                   r_ref):
    d2 = d2_ref[...]
    dist = jnp.sqrt(d2 + 1e-12)
    dp = jnp.clip(dist, 1e-2, 50.0)
    cent = lax.broadcasted_iota(jnp.int32, (1, 64), 1).astype(F32) * (10.0 / 63.0)
    t = dp - cent
    rbf = jnp.exp(-10.0 * t * t)
    h = (g1_ref[...] + g2_ref[...]
         + jnp.dot(rbf, wd_ref[...], preferred_element_type=F32)
         + jnp.dot(at_ref[...], wa_ref[...], preferred_element_type=F32)
         + bias_ref[...])
    r_ref[...] = jnp.maximum(h, 0.0)


def _edge_cov(g1, g2, d2, attr, wd, wattr, biasc, ep):
    eb = 512
    return pl.pallas_call(
        _edge_cov_body,
        grid=(ep // eb,),
        in_specs=[
            pl.BlockSpec((eb, D), lambda i: (i, 0)),
            pl.BlockSpec((eb, D), lambda i: (i, 0)),
            pl.BlockSpec((eb, 1), lambda i: (i, 0)),
            pl.BlockSpec((eb, 16), lambda i: (i, 0)),
            pl.BlockSpec((64, D), lambda i: (0, 0)),
            pl.BlockSpec((16, D), lambda i: (0, 0)),
            pl.BlockSpec((1, D), lambda i: (0, 0)),
        ],
        out_specs=pl.BlockSpec((eb, D), lambda i: (i, 0)),
        out_shape=jax.ShapeDtypeStruct((ep, D), F32),
    )(g1, g2, d2, attr, wd, wattr, biasc)


def _edge_ncov_body(g1_ref, g2_ref, d2_ref, wn_ref, bias_ref, r_ref):
    d2 = d2_ref[...]
    dist = jnp.sqrt(d2 + 1e-12)
    dc = jnp.clip(dist, 1e-2, 50.0)
    dp2 = 1.0 / (dc * dc)
    dp6 = dp2 * dp2 * dp2
    cent = lax.broadcasted_iota(jnp.int32, (1, 64), 1).astype(F32) * (10.0 / 63.0)
    t2 = dp2 - cent
    t6 = dp6 - cent
    rbf = jnp.concatenate([jnp.exp(-10.0 * t2 * t2),
                           jnp.exp(-10.0 * t6 * t6)], axis=1)
    h = (g1_ref[...] + g2_ref[...]
         + jnp.dot(rbf, wn_ref[...], preferred_element_type=F32)
         + bias_ref[...])
    r_ref[...] = jnp.maximum(h, 0.0)


def _edge_ncov(g1, g2, d2, wn, biasn, ep):
    eb = 512
    return pl.pallas_call(
        _edge_ncov_body,
        grid=(ep // eb,),
        in_specs=[
            pl.BlockSpec((eb, D), lambda i: (i, 0)),
            pl.BlockSpec((eb, D), lambda i: (i, 0)),
            pl.BlockSpec((eb, 1), lambda i: (i, 0)),
            pl.BlockSpec((D, D), lambda i: (0, 0)),
            pl.BlockSpec((1, D), lambda i: (0, 0)),
        ],
        out_specs=pl.BlockSpec((eb, D), lambda i: (i, 0)),
        out_shape=jax.ShapeDtypeStruct((ep, D), F32),
    )(g1, g2, d2, wn, biasn)


def _scatter(r, dst, ep):
    per = ep // NW
    chunks = per // CHUNK
    zr = NP // 16

    @functools.partial(
        pl.kernel,
        out_type=(jax.ShapeDtypeStruct((2, NP, D), F32),
                  jax.ShapeDtypeStruct((2, NP), F32)),
        mesh=_sc_mesh(),
        compiler_params=pltpu.CompilerParams(needs_layout_passes=False),
        scratch_types=[
            pltpu.VMEM((CHUNK,), jnp.int32),
            pltpu.VMEM((CHUNK, D), F32),
            pltpu.VMEM((NP,), F32),
            pltpu.VMEM((16, zr), F32),
            pltpu.VMEM((zr,), F32),
            pltpu.VMEM_SHARED((NP, D), F32),
            pltpu.VMEM_SHARED((16, NP), F32),
            pltpu.SemaphoreType.DMA,
            pltpu.SemaphoreType.DMA,
        ],
    )
    def sk(r_h, dst_h, zm_h, m_h, cnt_h,
           idx_v, rows_v, cnt_v, cbuf_v, cz_v, accm, cstage, s1, s2):
        cid = lax.axis_index("c")
        sid = lax.axis_index("s")
        wid = sid * 2 + cid
        pltpu.sync_copy(zm_h, accm.at[pl.ds(sid * zr, zr)])

        def zbody(i, carry):
            cnt_v[pl.ds(i * L, L)] = jnp.zeros((L,), F32)
            return carry

        lax.fori_loop(0, NP // L, zbody, 0)
        plsc.subcore_barrier()

        def body(j, carry):
            base = wid * per + j * CHUNK
            l1 = pltpu.async_copy(dst_h.at[pl.ds(base, CHUNK)], idx_v, s1)
            l2 = pltpu.async_copy(r_h.at[pl.ds(base, CHUNK)], rows_v, s2)
            l1.wait()
            l2.wait()
            pltpu.sync_copy(rows_v, accm.at[idx_v], add=True)
            for k in range(CHUNK // L):
                idx = idx_v[pl.ds(k * L, L)]
                plsc.addupdate_scatter(cnt_v, [idx], jnp.ones((L,), F32))
            return carry

        lax.fori_loop(0, chunks, body, 0)
        pltpu.sync_copy(cnt_v, cstage.at[sid])
        plsc.subcore_barrier()
        pltpu.sync_copy(accm.at[pl.ds(sid * zr, zr)],
                        m_h.at[cid, pl.ds(sid * zr, zr)])
        pltpu.sync_copy(cstage.at[:, pl.ds(sid * zr, zr)], cbuf_v)

        def rbody(g, carry):
            acc = jnp.zeros((L,), F32)
            for rr in range(16):
                acc = acc + cbuf_v[rr, pl.ds(g * L, L)]
            cz_v[pl.ds(g * L, L)] = acc
            return carry

        lax.fori_loop(0, zr // L, rbody, 0)
        pltpu.sync_copy(cz_v, cnt_h.at[cid, pl.ds(sid * zr, zr)])

    zm = jnp.zeros((zr, D), F32)
    return sk(r, dst, zm)


def _final_body(x_ref, mc_ref, cc0_ref, cc1_ref, mn_ref, cn0_ref, cn1_ref,
                wc2_ref, bc2_ref, wn2_ref, bn2_ref,
                wuc_ref, buc_ref, wun_ref, bun_ref, o_ref):
    xb = x_ref[...]
    sc = mc_ref[0] + mc_ref[1]
    cntc = cc0_ref[...] + cc1_ref[...]
    m_cov = (jnp.dot(sc, wc2_ref[...], preferred_element_type=F32)
             + cntc * bc2_ref[...])
    sn = mn_ref[0] + mn_ref[1]
    cntn = cn0_ref[...] + cn1_ref[...]
    m_ncov = (jnp.dot(sn, wn2_ref[...], preferred_element_type=F32)
              + cntn * bn2_ref[...])
    h_cov = jnp.maximum(
        jnp.dot(xb + m_cov, wuc_ref[...], preferred_element_type=F32)
        + buc_ref[...], 0.0)
    h_ncov = jnp.maximum(
        jnp.dot(xb + m_ncov, wun_ref[...], preferred_element_type=F32)
        + bun_ref[...], 0.0)
    o_ref[...] = h_cov + h_ncov


def _final(x, mc, cc0, cc1, mn, cn0, cn1,
           wc2, bc2r, wn2, bn2r, wuc, bucr, wun, bunr):
    n = x.shape[0]
    rb = 1000
    return pl.pallas_call(
        _final_body,
        grid=(n // rb,),
        in_specs=[
            pl.BlockSpec((rb, D), lambda i: (i, 0)),
            pl.BlockSpec((2, rb, D), lambda i: (0, i, 0)),
            pl.BlockSpec((rb, 1), lambda i: (i, 0)),
            pl.BlockSpec((rb, 1), lambda i: (i, 0)),
            pl.BlockSpec((2, rb, D), lambda i: (0, i, 0)),
            pl.BlockSpec((rb, 1), lambda i: (i, 0)),
            pl.BlockSpec((rb, 1), lambda i: (i, 0)),
            pl.BlockSpec((D, D), lambda i: (0, 0)),
            pl.BlockSpec((1, D), lambda i: (0, 0)),
            pl.BlockSpec((D, D), lambda i: (0, 0)),
            pl.BlockSpec((1, D), lambda i: (0, 0)),
            pl.BlockSpec((D, D), lambda i: (0, 0)),
            pl.BlockSpec((1, D), lambda i: (0, 0)),
            pl.BlockSpec((D, D), lambda i: (0, 0)),
            pl.BlockSpec((1, D), lambda i: (0, 0)),
        ],
        out_specs=pl.BlockSpec((rb, D), lambda i: (i, 0)),
        out_shape=jax.ShapeDtypeStruct((n, D), F32),
    )(x, mc, cc0, cc1, mn, cn0, cn1, wc2, bc2r, wn2, bn2r, wuc, bucr, wun, bunr)


def _pad_idx(idx, ep, fill):
    e = idx.shape[0]
    return jnp.concatenate([idx, jnp.full((ep - e,), fill, jnp.int32)])


def kernel(x, pos, edge_index_cov, edge_attr_cov, edge_index_ncov,
           W_bp, b_bp, Wc1, bc1, Wc2, bc2, Wn1, bn1, Wn2, bn2,
           Wuc, buc, Wun, bun):
    ec = edge_index_cov.shape[1]
    en = edge_index_ncov.shape[1]
    epc = -(-ec // (NW * CHUNK)) * (NW * CHUNK)
    epn = -(-en // (NW * CHUNK)) * (NW * CHUNK)

    wcat = jnp.concatenate([Wc1[0:128], Wc1[128:256], Wn1[0:128], Wn1[128:256]],
                           axis=1)
    t1c, t2c, t1n, t2n, wattr, biasc = _prep(
        x, wcat, W_bp, Wc1[256:384], b_bp[None, :], bc1[None, :])

    px, py, pz = pos[:, 0], pos[:, 1], pos[:, 2]
    rowc = _pad_idx(edge_index_cov[0], epc, 0)
    colc = _pad_idx(edge_index_cov[1], epc, 0)
    dstc = _pad_idx(edge_index_cov[1], epc, DUMMY)
    rown = _pad_idx(edge_index_ncov[0], epn, 0)
    coln = _pad_idx(edge_index_ncov[1], epn, 0)
    dstn = _pad_idx(edge_index_ncov[1], epn, DUMMY)
    attrp = jnp.pad(edge_attr_cov, ((0, epc - ec), (0, 0)))

    gc1, gc2, d2c = _gather(t1c, t2c, colc, rowc, px, py, pz, epc)
    gn1, gn2, d2n = _gather(t1n, t2n, coln, rown, px, py, pz, epn)

    rc = _edge_cov(gc1, gc2, d2c[:, None], attrp, Wc1[384:448], wattr, biasc,
                   epc)
    rn = _edge_ncov(gn1, gn2, d2n[:, None], Wn1[256:384], bn1[None, :], epn)

    mc, cc = _scatter(rc, dstc, epc)
    mn, cn = _scatter(rn, dstn, epn)

    out = _final(x, mc, cc[0][:, None], cc[1][:, None],
                 mn, cn[0][:, None], cn[1][:, None],
                 Wc2, bc2[None, :], Wn2, bn2[None, :],
                 Wuc, buc[None, :], Wun, bun[None, :])
    return out
